# baseline (speedup 1.0000x reference)
import functools

import jax
import jax.numpy as jnp
from jax import lax
from jax.experimental import pallas as pl
from jax.experimental.pallas import tpu as pltpu
from jax.experimental.pallas import tpu_sc as plsc

N = 10000
E = 320000
D = 128
K = 8
OUT_DIM = 128

DH = D // 2
NS = 16
NP = 10240
SLAB = NP // NS
CH = 128
NCHUNK = 2560
EP = NCHUNK * CH
CPT = NCHUNK // NS
NSLOT = 4
TRASH = NP + 64
GROWS = NP + 256
RB = 128
NB = SLAB // RB
YROWS = (K + 1) * NP


def _rsqrt16(d):
    bi = lax.bitcast_convert_type(d, jnp.int32)
    bi = jnp.int32(0x5F3759DF) - lax.shift_right_logical(bi, 1)
    x = lax.bitcast_convert_type(bi, jnp.float32)
    half = d * 0.5
    for _ in range(3):
        x = x * (1.5 - half * x * x)
    return x


def _sc_propagate(fab, row_p, col_p):
    mesh = plsc.VectorSubcoreMesh(
        core_axis_name="c", subcore_axis_name="s", num_cores=2
    )

    @functools.partial(
        pl.kernel,
        out_type=[
            jax.ShapeDtypeStruct((2 * YROWS, DH), jnp.float32),
            jax.ShapeDtypeStruct((2 * NP,), jnp.float32),
        ],
        mesh=mesh,
        compiler_params=pltpu.CompilerParams(use_tc_tiling_on_sc=False),
        scratch_types=[
            [pltpu.VMEM((CH, DH), jnp.float32) for _ in range(NSLOT)],
            [pltpu.VMEM((CH,), jnp.int32) for _ in range(NSLOT)],
            [pltpu.VMEM((CH,), jnp.int32) for _ in range(NSLOT)],
            pltpu.VMEM((CH,), jnp.float32),
            pltpu.VMEM((SLAB,), jnp.float32),
            pltpu.VMEM((SLAB,), jnp.float32),
            pltpu.VMEM((SLAB,), jnp.float32),
            pltpu.VMEM_SHARED((GROWS, DH), jnp.float32),
            pltpu.VMEM_SHARED((GROWS,), jnp.float32),
            [pltpu.SemaphoreType.DMA for _ in range(NSLOT)],
            [pltpu.SemaphoreType.DMA for _ in range(NSLOT)],
            [pltpu.SemaphoreType.DMA for _ in range(NSLOT)],
            [pltpu.SemaphoreType.DMA for _ in range(NSLOT)],
        ],
    )
    def k(fab_r, rows, cols, Yab, sinv, rows_b, idx_r, idx_c,
          ones_buf, deg_v, s_v, t_v, g, deg1, sem_i, sem_c, sem_g, sem_s):
        rows_buf0 = rows_b[0]
        cid = lax.axis_index("c")
        wid = lax.axis_index("s")
        nbase = wid * SLAB
        ybase = cid * YROWS

        def scale_slab(src_ref, src_base, scale_ref, dst_base):
            def block(bidx, _):
                base = bidx * RB
                pltpu.sync_copy(
                    src_ref.at[pl.ds(src_base + base, RB), :], rows_buf0
                )

                def grp(r16, _):
                    sv16 = scale_ref[pl.ds(bidx * RB + r16 * 16, 16)]
                    for ri in range(16):
                        sv = jnp.full((16,), sv16[ri], jnp.float32)
                        row = r16 * 16 + ri
                        for j in range(DH // 16):
                            sl = pl.ds(j * 16, 16)
                            rows_buf0[row, sl] = rows_buf0[row, sl] * sv
                    return 0
                lax.fori_loop(0, RB // 16, grp, 0)
                pltpu.sync_copy(
                    rows_buf0, Yab.at[pl.ds(dst_base + nbase + base, RB), :]
                )
                return 0
            lax.fori_loop(0, NB, block, 0)

        def fill16(r, _):
            ones_buf[pl.ds(r * 16, 16)] = jnp.full((16,), 1.0, jnp.float32)
            return 0
        lax.fori_loop(0, CH // 16, fill16, 0)

        def fill_slab(r, _):
            s_v[pl.ds(r * 16, 16)] = jnp.full((16,), 1.0, jnp.float32)
            return 0
        lax.fori_loop(0, SLAB // 16, fill_slab, 0)
        pltpu.sync_copy(s_v, deg1.at[pl.ds(nbase, SLAB)])
        plsc.subcore_barrier()

        def hist(gq, _):
            for b in range(2):
                j = 2 * gq + b

                @pl.when(gq >= 1)
                def _():
                    pltpu.make_async_copy(
                        ones_buf, deg1.at[idx_c[b]], sem_s[b]
                    ).wait()
                off = (wid + NS * j) * CH
                pltpu.sync_copy(cols.at[pl.ds(off, CH)], idx_c[b])
                pltpu.async_copy(ones_buf, deg1.at[idx_c[b]], sem_s[b],
                                 add=True)
            return 0
        lax.fori_loop(0, CPT // 2, hist, 0)
        for b in range(2):
            pltpu.make_async_copy(ones_buf, deg1.at[idx_c[b]], sem_s[b]).wait()
        plsc.subcore_barrier()

        pltpu.sync_copy(deg1.at[pl.ds(nbase, SLAB)], deg_v)

        def newton(r, _):
            sl = pl.ds(r * 16, 16)
            d = deg_v[sl]
            x = _rsqrt16(d)
            s_v[sl] = x
            t_v[sl] = x * x
            deg_v[sl] = d * x
            return 0
        lax.fori_loop(0, SLAB // 16, newton, 0)
        pltpu.sync_copy(deg_v, sinv.at[pl.ds(cid * NP + nbase, SLAB)])

        scale_slab(fab_r, cid * NP + nbase, s_v, ybase)
        plsc.subcore_barrier()

        def hop_body(h, _):
            src_off = ybase + h * NP

            def initblk(bidx, _):
                base = bidx * RB
                pltpu.sync_copy(
                    Yab.at[pl.ds(src_off + nbase + base, RB), :], rows_buf0
                )
                pltpu.sync_copy(rows_buf0, g.at[pl.ds(nbase + base, RB), :])
                return 0
            lax.fori_loop(0, NB, initblk, 0)
            plsc.subcore_barrier()

            ovec = jnp.full((16,), src_off, jnp.int32)

            def load_idx(jj, b):
                off = (wid + NS * jj) * CH
                pltpu.async_copy(rows.at[pl.ds(off, CH)], idx_r[b], sem_i[b])
                pltpu.async_copy(cols.at[pl.ds(off, CH)], idx_c[b], sem_c[b])

            def finish_idx(b):
                pltpu.make_async_copy(
                    rows.at[pl.ds(0, CH)], idx_r[b], sem_i[b]).wait()
                pltpu.make_async_copy(
                    cols.at[pl.ds(0, CH)], idx_c[b], sem_c[b]).wait()

                def addoff(q, _):
                    sl = pl.ds(q * 16, 16)
                    idx_r[b][sl] = idx_r[b][sl] + ovec
                    return 0
                lax.fori_loop(0, CH // 16, addoff, 0)

            load_idx(0, 0)
            finish_idx(0)
            pltpu.async_copy(Yab.at[idx_r[0]], rows_b[0], sem_g[0])

            def edge_quad(gq, _):
                for u in range(NSLOT):
                    j = gq * NSLOT + u
                    b = u
                    bn = (u + 1) % NSLOT

                    @pl.when(j >= NSLOT - 1)
                    def _():
                        pltpu.make_async_copy(
                            rows_b[bn], g.at[idx_c[bn]], sem_s[bn]).wait()

                    @pl.when(j + 1 < CPT)
                    def _():
                        load_idx(j + 1, bn)
                    pltpu.make_async_copy(
                        Yab.at[idx_r[b]], rows_b[b], sem_g[b]).wait()
                    pltpu.async_copy(rows_b[b], g.at[idx_c[b]], sem_s[b],
                                     add=True)

                    @pl.when(j + 1 < CPT)
                    def _():
                        finish_idx(bn)
                        pltpu.async_copy(
                            Yab.at[idx_r[bn]], rows_b[bn], sem_g[bn])
                return 0
            lax.fori_loop(0, CPT // NSLOT, edge_quad, 0)
            for b in [(CPT - 3) % NSLOT, (CPT - 2) % NSLOT, (CPT - 1) % NSLOT]:
                pltpu.make_async_copy(rows_b[b], g.at[idx_c[b]],
                                      sem_s[b]).wait()
            plsc.subcore_barrier()

            scale_slab(g, nbase, t_v, src_off + NP)
            plsc.subcore_barrier()
            return 0
        lax.fori_loop(0, K, hop_body, 0)

    return k(fab, row_p, col_p)


def _tc_linear(YA, YB, WA9, WB9, sinv_col, b):
    nb = NP // 512

    def mm(ya_ref, yb_ref, wa_ref, wb_ref, sv_ref, b_ref, o_ref):
        kk = pl.program_id(1)

        @pl.when(kk == 0)
        def _():
            o_ref[...] = jnp.zeros_like(o_ref)

        o_ref[...] += lax.dot_general(
            ya_ref[0], wa_ref[0], (((1,), (1,)), ((), ())),
            preferred_element_type=jnp.float32,
        ) + lax.dot_general(
            yb_ref[0], wb_ref[0], (((1,), (1,)), ((), ())),
            preferred_element_type=jnp.float32,
        )

        @pl.when(kk == K)
        def _():
            o_ref[...] = o_ref[...] * sv_ref[...] + b_ref[...]

    return pl.pallas_call(
        mm,
        grid=(nb, K + 1),
        in_specs=[
            pl.BlockSpec((1, 512, DH), lambda i, kq: (kq, i, 0)),
            pl.BlockSpec((1, 512, DH), lambda i, kq: (kq, i, 0)),
            pl.BlockSpec((1, OUT_DIM, DH), lambda i, kq: (kq, 0, 0)),
            pl.BlockSpec((1, OUT_DIM, DH), lambda i, kq: (kq, 0, 0)),
            pl.BlockSpec((512, 1), lambda i, kq: (i, 0)),
            pl.BlockSpec((1, OUT_DIM), lambda i, kq: (0, 0)),
        ],
        out_specs=pl.BlockSpec((512, OUT_DIM), lambda i, kq: (i, 0)),
        out_shape=jax.ShapeDtypeStruct((NP, OUT_DIM), jnp.float32),
    )(YA, YB, WA9, WB9, sinv_col, b)


@jax.jit
def kernel(feature, edge_index, W, b):
    fa = jnp.pad(feature[:, :DH], ((0, NP - N), (0, 0)))
    fb = jnp.pad(feature[:, DH:], ((0, NP - N), (0, 0)))
    fab = jnp.concatenate([fa, fb], axis=0)
    row = edge_index[0]
    col = edge_index[1]
    row_p = jnp.pad(row, (0, EP - E))
    col_p = jnp.pad(col, (0, EP - E), constant_values=TRASH)

    Yab, sinv = _sc_propagate(fab, row_p, col_p)
    YA = Yab[:YROWS].reshape(K + 1, NP, DH)
    YB = Yab[YROWS:].reshape(K + 1, NP, DH)

    W9 = jnp.transpose(W.reshape(OUT_DIM, K + 1, D), (1, 0, 2))
    out = _tc_linear(YA, YB, W9[:, :, :DH], W9[:, :, DH:],
                     sinv[:NP].reshape(NP, 1), b.reshape(1, OUT_DIM))
    return out[:N]


# speedup vs baseline: 1.4926x; 1.4926x over previous
import functools

import jax
import jax.numpy as jnp
from jax import lax
from jax.experimental import pallas as pl
from jax.experimental.pallas import tpu as pltpu
from jax.experimental.pallas import tpu_sc as plsc

N = 10000
E = 320000
D = 128
K = 8
OUT_DIM = 128

DH = D // 2
NS = 16
NP = 10240
SLAB = NP // NS
CH = 128
NCHUNK = 2528
EP = NCHUNK * CH
CPT = NCHUNK // NS
TRASH = NP + 64
GROWS = NP + 256
RB = 128
NB = SLAB // RB
YROWS = (K + 1) * NP


def _rsqrt16(d):
    bi = lax.bitcast_convert_type(d, jnp.int32)
    bi = jnp.int32(0x5F3759DF) - lax.shift_right_logical(bi, 1)
    x = lax.bitcast_convert_type(bi, jnp.float32)
    half = d * 0.5
    for _ in range(3):
        x = x * (1.5 - half * x * x)
    return x


def _sc_propagate(fab, row_p, col_p):
    mesh = plsc.VectorSubcoreMesh(
        core_axis_name="c", subcore_axis_name="s", num_cores=2
    )

    @functools.partial(
        pl.kernel,
        out_type=[
            jax.ShapeDtypeStruct((2 * YROWS, DH), jnp.float32),
            jax.ShapeDtypeStruct((2 * NP,), jnp.float32),
        ],
        mesh=mesh,
        compiler_params=pltpu.CompilerParams(use_tc_tiling_on_sc=False),
        scratch_types=[
            pltpu.VMEM((CH, DH), jnp.float32),
            pltpu.VMEM((CH, DH), jnp.float32),
            pltpu.VMEM((CH,), jnp.int32),
            pltpu.VMEM((CH,), jnp.int32),
            pltpu.VMEM((CH,), jnp.int32),
            pltpu.VMEM((CH,), jnp.int32),
            pltpu.VMEM((CH,), jnp.float32),
            pltpu.VMEM((SLAB,), jnp.float32),
            pltpu.VMEM((SLAB,), jnp.float32),
            pltpu.VMEM((SLAB,), jnp.float32),
            pltpu.VMEM_SHARED((GROWS, DH), jnp.float32),
            pltpu.VMEM_SHARED((NP, DH), jnp.float32),
            pltpu.VMEM_SHARED((GROWS,), jnp.float32),
            pltpu.SemaphoreType.DMA,
            pltpu.SemaphoreType.DMA,
            pltpu.SemaphoreType.DMA,
            pltpu.SemaphoreType.DMA,
            pltpu.SemaphoreType.DMA,
            pltpu.SemaphoreType.DMA,
            pltpu.SemaphoreType.DMA,
            pltpu.SemaphoreType.DMA,
        ],
    )
    def k(fab_r, rows, cols, Yab, sinv, rows_buf0, rows_buf1, idx_row0,
          idx_row1, idx_col0, idx_col1, ones_buf, deg_v, s_v, t_v, g, y_sp,
          deg1, sem_i0, sem_i1, sem_c0, sem_c1, sem_g0, sem_g1, sem_s0,
          sem_s1):
        rows_b = (rows_buf0, rows_buf1)
        idx_r = (idx_row0, idx_row1)
        idx_c = (idx_col0, idx_col1)
        sem_i = (sem_i0, sem_i1)
        sem_c = (sem_c0, sem_c1)
        sem_g = (sem_g0, sem_g1)
        sem_s = (sem_s0, sem_s1)
        cid = lax.axis_index("c")
        wid = lax.axis_index("s")
        nbase = wid * SLAB
        ybase = cid * YROWS

        def scale_slab(src_ref, src_base, scale_ref, dst_base):
            def block(bidx, _):
                base = bidx * RB
                pltpu.sync_copy(
                    src_ref.at[pl.ds(src_base + base, RB), :], rows_buf0
                )

                def grp(r16, _):
                    sv16 = scale_ref[pl.ds(bidx * RB + r16 * 16, 16)]
                    for ri in range(16):
                        sv = jnp.full((16,), sv16[ri], jnp.float32)
                        row = r16 * 16 + ri
                        for j in range(DH // 16):
                            sl = pl.ds(j * 16, 16)
                            rows_buf0[row, sl] = rows_buf0[row, sl] * sv
                    return 0
                lax.fori_loop(0, RB // 16, grp, 0)
                pltpu.sync_copy(
                    rows_buf0, y_sp.at[pl.ds(nbase + base, RB), :]
                )
                pltpu.sync_copy(
                    rows_buf0, Yab.at[pl.ds(dst_base + nbase + base, RB), :]
                )
                return 0
            lax.fori_loop(0, NB, block, 0)

        def fill16(r, _):
            ones_buf[pl.ds(r * 16, 16)] = jnp.full((16,), 1.0, jnp.float32)
            return 0
        lax.fori_loop(0, CH // 16, fill16, 0)

        def fill_slab(r, _):
            s_v[pl.ds(r * 16, 16)] = jnp.full((16,), 1.0, jnp.float32)
            return 0
        lax.fori_loop(0, SLAB // 16, fill_slab, 0)
        pltpu.sync_copy(s_v, deg1.at[pl.ds(nbase, SLAB)])
        plsc.subcore_barrier()

        def hist(gq, _):
            for b in range(2):
                j = 2 * gq + b

                @pl.when(gq >= 1)
                def _():
                    pltpu.make_async_copy(
                        ones_buf, deg1.at[idx_c[b]], sem_s[b]
                    ).wait()
                off = (wid + NS * j) * CH
                pltpu.sync_copy(cols.at[pl.ds(off, CH)], idx_c[b])
                pltpu.async_copy(ones_buf, deg1.at[idx_c[b]], sem_s[b],
                                 add=True)
            return 0
        lax.fori_loop(0, CPT // 2, hist, 0)
        for b in range(2):
            pltpu.make_async_copy(ones_buf, deg1.at[idx_c[b]], sem_s[b]).wait()
        plsc.subcore_barrier()

        pltpu.sync_copy(deg1.at[pl.ds(nbase, SLAB)], deg_v)

        def newton(r, _):
            sl = pl.ds(r * 16, 16)
            d = deg_v[sl]
            x = _rsqrt16(d)
            s_v[sl] = x
            t_v[sl] = x * x
            deg_v[sl] = d * x
            return 0
        lax.fori_loop(0, SLAB // 16, newton, 0)
        pltpu.sync_copy(deg_v, sinv.at[pl.ds(cid * NP + nbase, SLAB)])

        scale_slab(fab_r, cid * NP + nbase, s_v, ybase)
        plsc.subcore_barrier()

        def hop_body(h, _):
            src_off = ybase + h * NP

            def initblk(bidx, _):
                base = bidx * RB
                pltpu.sync_copy(
                    y_sp.at[pl.ds(nbase + base, RB), :], rows_buf0
                )
                pltpu.sync_copy(rows_buf0, g.at[pl.ds(nbase + base, RB), :])
                return 0
            lax.fori_loop(0, NB, initblk, 0)
            plsc.subcore_barrier()

            def load_idx(jj, b):
                off = (wid + NS * jj) * CH
                pltpu.async_copy(rows.at[pl.ds(off, CH)], idx_r[b], sem_i[b])
                pltpu.async_copy(cols.at[pl.ds(off, CH)], idx_c[b], sem_c[b])

            def finish_idx(b):
                pltpu.make_async_copy(
                    rows.at[pl.ds(0, CH)], idx_r[b], sem_i[b]).wait()
                pltpu.make_async_copy(
                    cols.at[pl.ds(0, CH)], idx_c[b], sem_c[b]).wait()

            for b in range(2):
                load_idx(b, b)
                finish_idx(b)
                pltpu.async_copy(y_sp.at[idx_r[b]], rows_b[b], sem_g[b])

            def edge_pair(gq, _):
                for b in range(2):
                    j = 2 * gq + b
                    pltpu.make_async_copy(
                        y_sp.at[idx_r[b]], rows_b[b], sem_g[b]).wait()
                    pltpu.async_copy(rows_b[b], g.at[idx_c[b]], sem_s[b],
                                     add=True)
                    pltpu.make_async_copy(
                        rows_b[b], g.at[idx_c[b]], sem_s[b]).wait()

                    @pl.when(j + 2 < CPT)
                    def _():
                        load_idx(j + 2, b)
                        finish_idx(b)
                        pltpu.async_copy(y_sp.at[idx_r[b]], rows_b[b],
                                         sem_g[b])
                return 0
            lax.fori_loop(0, CPT // 2, edge_pair, 0)
            plsc.subcore_barrier()

            scale_slab(g, nbase, t_v, src_off + NP)
            plsc.subcore_barrier()
            return 0
        lax.fori_loop(0, K, hop_body, 0)

    return k(fab, row_p, col_p)


def _tc_linear(YA, YB, WA9, WB9, sinv_col, b):
    nb = NP // 512

    def mm(ya_ref, yb_ref, wa_ref, wb_ref, sv_ref, b_ref, o_ref):
        kk = pl.program_id(1)

        @pl.when(kk == 0)
        def _():
            o_ref[...] = jnp.zeros_like(o_ref)

        o_ref[...] += lax.dot_general(
            ya_ref[0], wa_ref[0], (((1,), (1,)), ((), ())),
            preferred_element_type=jnp.float32,
        ) + lax.dot_general(
            yb_ref[0], wb_ref[0], (((1,), (1,)), ((), ())),
            preferred_element_type=jnp.float32,
        )

        @pl.when(kk == K)
        def _():
            o_ref[...] = o_ref[...] * sv_ref[...] + b_ref[...]

    return pl.pallas_call(
        mm,
        grid=(nb, K + 1),
        in_specs=[
            pl.BlockSpec((1, 512, DH), lambda i, kq: (kq, i, 0)),
            pl.BlockSpec((1, 512, DH), lambda i, kq: (kq, i, 0)),
            pl.BlockSpec((1, OUT_DIM, DH), lambda i, kq: (kq, 0, 0)),
            pl.BlockSpec((1, OUT_DIM, DH), lambda i, kq: (kq, 0, 0)),
            pl.BlockSpec((512, 1), lambda i, kq: (i, 0)),
            pl.BlockSpec((1, OUT_DIM), lambda i, kq: (0, 0)),
        ],
        out_specs=pl.BlockSpec((512, OUT_DIM), lambda i, kq: (i, 0)),
        out_shape=jax.ShapeDtypeStruct((NP, OUT_DIM), jnp.float32),
    )(YA, YB, WA9, WB9, sinv_col, b)


@jax.jit
def kernel(feature, edge_index, W, b):
    fa = jnp.pad(feature[:, :DH], ((0, NP - N), (0, 0)))
    fb = jnp.pad(feature[:, DH:], ((0, NP - N), (0, 0)))
    fab = jnp.concatenate([fa, fb], axis=0)
    row = edge_index[0]
    col = edge_index[1]
    row_p = jnp.pad(row, (0, EP - E))
    col_p = jnp.pad(col, (0, EP - E), constant_values=TRASH)

    Yab, sinv = _sc_propagate(fab, row_p, col_p)
    YA = Yab[:YROWS].reshape(K + 1, NP, DH)
    YB = Yab[YROWS:].reshape(K + 1, NP, DH)

    W9 = jnp.transpose(W.reshape(OUT_DIM, K + 1, D), (1, 0, 2))
    out = _tc_linear(YA, YB, W9[:, :, :DH], W9[:, :, DH:],
                     sinv[:NP].reshape(NP, 1), b.reshape(1, OUT_DIM))
    return out[:N]


# speedup vs baseline: 1.5049x; 1.0082x over previous
import functools

import jax
import jax.numpy as jnp
from jax import lax
from jax.experimental import pallas as pl
from jax.experimental.pallas import tpu as pltpu
from jax.experimental.pallas import tpu_sc as plsc

N = 10000
E = 320000
D = 128
K = 8
OUT_DIM = 128

DH = D // 2
NS = 16
NP = 10240
SLAB = NP // NS
CH = 128
NCHUNK = 2528
EP = NCHUNK * CH
CPT = NCHUNK // NS
TRASH = NP + 64
GROWS = NP + 256
RB = 128
NB = SLAB // RB
YROWS = (K + 1) * NP


def _rsqrt16(d):
    bi = lax.bitcast_convert_type(d, jnp.int32)
    bi = jnp.int32(0x5F3759DF) - lax.shift_right_logical(bi, 1)
    x = lax.bitcast_convert_type(bi, jnp.float32)
    half = d * 0.5
    for _ in range(3):
        x = x * (1.5 - half * x * x)
    return x


def _sc_propagate(fab, row_p, col_p):
    mesh = plsc.VectorSubcoreMesh(
        core_axis_name="c", subcore_axis_name="s", num_cores=2
    )

    @functools.partial(
        pl.kernel,
        out_type=[
            jax.ShapeDtypeStruct((2 * YROWS, DH), jnp.float32),
            jax.ShapeDtypeStruct((2 * NP,), jnp.float32),
        ],
        mesh=mesh,
        compiler_params=pltpu.CompilerParams(use_tc_tiling_on_sc=False),
        scratch_types=[
            pltpu.VMEM((CH, DH), jnp.float32),
            pltpu.VMEM((CH, DH), jnp.float32),
            pltpu.VMEM((CH,), jnp.int32),
            pltpu.VMEM((CH,), jnp.int32),
            pltpu.VMEM((CH,), jnp.int32),
            pltpu.VMEM((CH,), jnp.int32),
            pltpu.VMEM((CH,), jnp.float32),
            pltpu.VMEM((SLAB,), jnp.float32),
            pltpu.VMEM((SLAB,), jnp.float32),
            pltpu.VMEM((SLAB,), jnp.float32),
            pltpu.VMEM_SHARED((GROWS, DH), jnp.float32),
            pltpu.VMEM_SHARED((NP, DH), jnp.float32),
            pltpu.VMEM_SHARED((GROWS,), jnp.float32),
            pltpu.SemaphoreType.DMA,
            pltpu.SemaphoreType.DMA,
            pltpu.SemaphoreType.DMA,
            pltpu.SemaphoreType.DMA,
            pltpu.SemaphoreType.DMA,
            pltpu.SemaphoreType.DMA,
            pltpu.SemaphoreType.DMA,
            pltpu.SemaphoreType.DMA,
        ],
    )
    def k(fab_r, rows, cols, Yab, sinv, rows_buf0, rows_buf1, idx_row0,
          idx_row1, idx_col0, idx_col1, ones_buf, deg_v, s_v, t_v, g, y_sp,
          deg1, sem_i0, sem_i1, sem_c0, sem_c1, sem_g0, sem_g1, sem_s0,
          sem_s1):
        rows_b = (rows_buf0, rows_buf1)
        idx_r = (idx_row0, idx_row1)
        idx_c = (idx_col0, idx_col1)
        sem_i = (sem_i0, sem_i1)
        sem_c = (sem_c0, sem_c1)
        sem_g = (sem_g0, sem_g1)
        sem_s = (sem_s0, sem_s1)
        cid = lax.axis_index("c")
        wid = lax.axis_index("s")
        nbase = wid * SLAB
        ybase = cid * YROWS

        def scale_slab(src_ref, src_base, scale_ref, dst_base):
            def wr_descs(bidx, b):
                base = bidx * RB
                return (
                    pltpu.make_async_copy(
                        rows_b[b], y_sp.at[pl.ds(nbase + base, RB), :],
                        sem_g[b]),
                    pltpu.make_async_copy(
                        rows_b[b],
                        Yab.at[pl.ds(dst_base + nbase + base, RB), :],
                        sem_s[b]),
                )

            for bidx in range(NB):
                b = bidx % 2
                if bidx >= 2:
                    for dsc in wr_descs(bidx - 2, b):
                        dsc.wait()
                base = bidx * RB
                pltpu.sync_copy(
                    src_ref.at[pl.ds(src_base + base, RB), :], rows_b[b]
                )

                def grp(r16, _, bidx=bidx, b=b):
                    sv16 = scale_ref[pl.ds(bidx * RB + r16 * 16, 16)]
                    for ri in range(16):
                        sv = jnp.full((16,), sv16[ri], jnp.float32)
                        row = r16 * 16 + ri
                        for j in range(DH // 16):
                            sl = pl.ds(j * 16, 16)
                            rows_b[b][row, sl] = rows_b[b][row, sl] * sv
                    return 0
                lax.fori_loop(0, RB // 16, grp, 0)
                pltpu.async_copy(
                    rows_b[b], y_sp.at[pl.ds(nbase + base, RB), :], sem_g[b])
                pltpu.async_copy(
                    rows_b[b], Yab.at[pl.ds(dst_base + nbase + base, RB), :],
                    sem_s[b])
            for bidx in (NB - 2, NB - 1):
                for dsc in wr_descs(bidx, bidx % 2):
                    dsc.wait()

        def fill16(r, _):
            ones_buf[pl.ds(r * 16, 16)] = jnp.full((16,), 1.0, jnp.float32)
            return 0
        lax.fori_loop(0, CH // 16, fill16, 0)

        def fill_slab(r, _):
            s_v[pl.ds(r * 16, 16)] = jnp.full((16,), 1.0, jnp.float32)
            return 0
        lax.fori_loop(0, SLAB // 16, fill_slab, 0)
        pltpu.sync_copy(s_v, deg1.at[pl.ds(nbase, SLAB)])
        plsc.subcore_barrier()

        def hist(gq, _):
            for b in range(2):
                j = 2 * gq + b

                @pl.when(gq >= 1)
                def _():
                    pltpu.make_async_copy(
                        ones_buf, deg1.at[idx_c[b]], sem_s[b]
                    ).wait()
                off = (wid + NS * j) * CH
                pltpu.sync_copy(cols.at[pl.ds(off, CH)], idx_c[b])
                pltpu.async_copy(ones_buf, deg1.at[idx_c[b]], sem_s[b],
                                 add=True)
            return 0
        lax.fori_loop(0, CPT // 2, hist, 0)
        for b in range(2):
            pltpu.make_async_copy(ones_buf, deg1.at[idx_c[b]], sem_s[b]).wait()
        plsc.subcore_barrier()

        pltpu.sync_copy(deg1.at[pl.ds(nbase, SLAB)], deg_v)

        def newton(r, _):
            sl = pl.ds(r * 16, 16)
            d = deg_v[sl]
            x = _rsqrt16(d)
            s_v[sl] = x
            t_v[sl] = x * x
            deg_v[sl] = d * x
            return 0
        lax.fori_loop(0, SLAB // 16, newton, 0)
        pltpu.sync_copy(deg_v, sinv.at[pl.ds(cid * NP + nbase, SLAB)])

        scale_slab(fab_r, cid * NP + nbase, s_v, ybase)
        plsc.subcore_barrier()

        def hop_body(h, _):
            src_off = ybase + h * NP

            def g_desc(bidx, b):
                base = bidx * RB
                return pltpu.make_async_copy(
                    rows_b[b], g.at[pl.ds(nbase + base, RB), :], sem_s[b])

            for bidx in range(NB):
                b = bidx % 2
                if bidx >= 2:
                    g_desc(bidx - 2, b).wait()
                base = bidx * RB
                pltpu.sync_copy(
                    y_sp.at[pl.ds(nbase + base, RB), :], rows_b[b]
                )
                pltpu.async_copy(
                    rows_b[b], g.at[pl.ds(nbase + base, RB), :], sem_s[b])
            for bidx in (NB - 2, NB - 1):
                g_desc(bidx, bidx % 2).wait()
            plsc.subcore_barrier()

            def load_idx(jj, b):
                off = (wid + NS * jj) * CH
                pltpu.async_copy(rows.at[pl.ds(off, CH)], idx_r[b], sem_i[b])
                pltpu.async_copy(cols.at[pl.ds(off, CH)], idx_c[b], sem_c[b])

            def finish_idx(b):
                pltpu.make_async_copy(
                    rows.at[pl.ds(0, CH)], idx_r[b], sem_i[b]).wait()
                pltpu.make_async_copy(
                    cols.at[pl.ds(0, CH)], idx_c[b], sem_c[b]).wait()

            for b in range(2):
                load_idx(b, b)
                finish_idx(b)
                pltpu.async_copy(y_sp.at[idx_r[b]], rows_b[b], sem_g[b])

            def edge_pair(gq, _):
                for b in range(2):
                    j = 2 * gq + b
                    pltpu.make_async_copy(
                        y_sp.at[idx_r[b]], rows_b[b], sem_g[b]).wait()
                    pltpu.async_copy(rows_b[b], g.at[idx_c[b]], sem_s[b],
                                     add=True)
                    pltpu.make_async_copy(
                        rows_b[b], g.at[idx_c[b]], sem_s[b]).wait()

                    @pl.when(j + 2 < CPT)
                    def _():
                        load_idx(j + 2, b)
                        finish_idx(b)
                        pltpu.async_copy(y_sp.at[idx_r[b]], rows_b[b],
                                         sem_g[b])
                return 0
            lax.fori_loop(0, CPT // 2, edge_pair, 0)
            plsc.subcore_barrier()

            scale_slab(g, nbase, t_v, src_off + NP)
            plsc.subcore_barrier()
            return 0
        lax.fori_loop(0, K, hop_body, 0)

    return k(fab, row_p, col_p)


def _tc_linear(YA, YB, WA9, WB9, sinv_col, b):
    nb = NP // 512

    def mm(ya_ref, yb_ref, wa_ref, wb_ref, sv_ref, b_ref, o_ref):
        kk = pl.program_id(1)

        @pl.when(kk == 0)
        def _():
            o_ref[...] = jnp.zeros_like(o_ref)

        o_ref[...] += lax.dot_general(
            ya_ref[0], wa_ref[0], (((1,), (1,)), ((), ())),
            preferred_element_type=jnp.float32,
        ) + lax.dot_general(
            yb_ref[0], wb_ref[0], (((1,), (1,)), ((), ())),
            preferred_element_type=jnp.float32,
        )

        @pl.when(kk == K)
        def _():
            o_ref[...] = o_ref[...] * sv_ref[...] + b_ref[...]

    return pl.pallas_call(
        mm,
        grid=(nb, K + 1),
        in_specs=[
            pl.BlockSpec((1, 512, DH), lambda i, kq: (kq, i, 0)),
            pl.BlockSpec((1, 512, DH), lambda i, kq: (kq, i, 0)),
            pl.BlockSpec((1, OUT_DIM, DH), lambda i, kq: (kq, 0, 0)),
            pl.BlockSpec((1, OUT_DIM, DH), lambda i, kq: (kq, 0, 0)),
            pl.BlockSpec((512, 1), lambda i, kq: (i, 0)),
            pl.BlockSpec((1, OUT_DIM), lambda i, kq: (0, 0)),
        ],
        out_specs=pl.BlockSpec((512, OUT_DIM), lambda i, kq: (i, 0)),
        out_shape=jax.ShapeDtypeStruct((NP, OUT_DIM), jnp.float32),
    )(YA, YB, WA9, WB9, sinv_col, b)


@jax.jit
def kernel(feature, edge_index, W, b):
    fa = jnp.pad(feature[:, :DH], ((0, NP - N), (0, 0)))
    fb = jnp.pad(feature[:, DH:], ((0, NP - N), (0, 0)))
    fab = jnp.concatenate([fa, fb], axis=0)
    row = edge_index[0]
    col = edge_index[1]
    row_p = jnp.pad(row, (0, EP - E))
    col_p = jnp.pad(col, (0, EP - E), constant_values=TRASH)

    Yab, sinv = _sc_propagate(fab, row_p, col_p)
    YA = Yab[:YROWS].reshape(K + 1, NP, DH)
    YB = Yab[YROWS:].reshape(K + 1, NP, DH)

    W9 = jnp.transpose(W.reshape(OUT_DIM, K + 1, D), (1, 0, 2))
    out = _tc_linear(YA, YB, W9[:, :, :DH], W9[:, :, DH:],
                     sinv[:NP].reshape(NP, 1), b.reshape(1, OUT_DIM))
    return out[:N]


# speedup vs baseline: 1.9161x; 1.2732x over previous
import functools

import jax
import jax.numpy as jnp
from jax import lax
from jax.experimental import pallas as pl
from jax.experimental.pallas import tpu as pltpu
from jax.experimental.pallas import tpu_sc as plsc

N = 10000
E = 320000
D = 128
K = 8
OUT_DIM = 128

DH = D // 2
NS = 16
NP = 10240
SLAB = NP // NS
CH = 128
NCHUNK = 2560
EP = NCHUNK * CH
CPT = NCHUNK // NS
TRASH = NP + 64
GROWS = NP + 256
RB = 128
NB = SLAB // RB
YROWS = (K + 1) * NP


def _rsqrt16(d):
    bi = lax.bitcast_convert_type(d, jnp.int32)
    bi = jnp.int32(0x5F3759DF) - lax.shift_right_logical(bi, 1)
    x = lax.bitcast_convert_type(bi, jnp.float32)
    half = d * 0.5
    for _ in range(3):
        x = x * (1.5 - half * x * x)
    return x


def _sc_propagate(fab, row_p, col_p):
    mesh = plsc.VectorSubcoreMesh(
        core_axis_name="c", subcore_axis_name="s", num_cores=2
    )

    @functools.partial(
        pl.kernel,
        out_type=[
            jax.ShapeDtypeStruct((2 * YROWS, DH), jnp.float32),
            jax.ShapeDtypeStruct((2 * NP,), jnp.float32),
        ],
        mesh=mesh,
        compiler_params=pltpu.CompilerParams(use_tc_tiling_on_sc=False),
        scratch_types=[
            pltpu.VMEM((CH, DH), jnp.float32),
            pltpu.VMEM((CH, DH), jnp.float32),
            [pltpu.VMEM((CH,), jnp.int32) for _ in range(4)],
            [pltpu.VMEM((CH,), jnp.int32) for _ in range(4)],
            pltpu.VMEM((CH,), jnp.float32),
            pltpu.VMEM((SLAB,), jnp.float32),
            pltpu.VMEM((SLAB,), jnp.float32),
            pltpu.VMEM((SLAB,), jnp.float32),
            pltpu.VMEM_SHARED((GROWS, DH), jnp.float32),
            pltpu.VMEM_SHARED((NP, DH), jnp.float32),
            pltpu.VMEM_SHARED((GROWS,), jnp.float32),
            [pltpu.SemaphoreType.DMA for _ in range(4)],
            [pltpu.SemaphoreType.DMA for _ in range(4)],
            pltpu.SemaphoreType.DMA,
            pltpu.SemaphoreType.DMA,
            pltpu.SemaphoreType.DMA,
            pltpu.SemaphoreType.DMA,
        ],
    )
    def k(fab_r, rows, cols, Yab, sinv, rows_buf0, rows_buf1, idx_r, idx_c,
          ones_buf, deg_v, s_v, t_v, g, y_sp, deg1, sem_i, sem_c,
          sem_g0, sem_g1, sem_s0, sem_s1):
        rows_b = (rows_buf0, rows_buf1)
        sem_g = (sem_g0, sem_g1)
        sem_s = (sem_s0, sem_s1)
        cid = lax.axis_index("c")
        wid = lax.axis_index("s")
        nbase = wid * SLAB
        ybase = cid * YROWS

        def scale_slab(src_ref, src_base, scale_ref, dst_base):
            def wr_descs(bidx, b):
                base = bidx * RB
                return (
                    pltpu.make_async_copy(
                        rows_b[b], y_sp.at[pl.ds(nbase + base, RB), :],
                        sem_g[b]),
                    pltpu.make_async_copy(
                        rows_b[b],
                        Yab.at[pl.ds(dst_base + nbase + base, RB), :],
                        sem_s[b]),
                )

            for bidx in range(NB):
                b = bidx % 2
                if bidx >= 2:
                    for dsc in wr_descs(bidx - 2, b):
                        dsc.wait()
                base = bidx * RB
                pltpu.sync_copy(
                    src_ref.at[pl.ds(src_base + base, RB), :], rows_b[b]
                )

                def grp(r16, _, bidx=bidx, b=b):
                    sv16 = scale_ref[pl.ds(bidx * RB + r16 * 16, 16)]
                    for ri in range(16):
                        sv = jnp.full((16,), sv16[ri], jnp.float32)
                        row = r16 * 16 + ri
                        for j in range(DH // 16):
                            sl = pl.ds(j * 16, 16)
                            rows_b[b][row, sl] = rows_b[b][row, sl] * sv
                    return 0
                lax.fori_loop(0, RB // 16, grp, 0)
                pltpu.async_copy(
                    rows_b[b], y_sp.at[pl.ds(nbase + base, RB), :], sem_g[b])
                pltpu.async_copy(
                    rows_b[b], Yab.at[pl.ds(dst_base + nbase + base, RB), :],
                    sem_s[b])
            for bidx in (NB - 2, NB - 1):
                for dsc in wr_descs(bidx, bidx % 2):
                    dsc.wait()

        def fill16(r, _):
            ones_buf[pl.ds(r * 16, 16)] = jnp.full((16,), 1.0, jnp.float32)
            return 0
        lax.fori_loop(0, CH // 16, fill16, 0)

        def fill_slab(r, _):
            s_v[pl.ds(r * 16, 16)] = jnp.full((16,), 1.0, jnp.float32)
            return 0
        lax.fori_loop(0, SLAB // 16, fill_slab, 0)
        pltpu.sync_copy(s_v, deg1.at[pl.ds(nbase, SLAB)])
        plsc.subcore_barrier()

        def hist(gq, _):
            for b in range(2):
                j = 2 * gq + b

                @pl.when(gq >= 1)
                def _():
                    pltpu.make_async_copy(
                        ones_buf, deg1.at[idx_c[b]], sem_s[b]
                    ).wait()
                off = (wid + NS * j) * CH
                pltpu.sync_copy(cols.at[pl.ds(off, CH)], idx_c[b])
                pltpu.async_copy(ones_buf, deg1.at[idx_c[b]], sem_s[b],
                                 add=True)
            return 0
        lax.fori_loop(0, CPT // 2, hist, 0)
        for b in range(2):
            pltpu.make_async_copy(ones_buf, deg1.at[idx_c[b]], sem_s[b]).wait()
        plsc.subcore_barrier()

        pltpu.sync_copy(deg1.at[pl.ds(nbase, SLAB)], deg_v)

        def newton(r, _):
            sl = pl.ds(r * 16, 16)
            d = deg_v[sl]
            x = _rsqrt16(d)
            s_v[sl] = x
            t_v[sl] = x * x
            deg_v[sl] = d * x
            return 0
        lax.fori_loop(0, SLAB // 16, newton, 0)
        pltpu.sync_copy(deg_v, sinv.at[pl.ds(cid * NP + nbase, SLAB)])

        scale_slab(fab_r, cid * NP + nbase, s_v, ybase)
        plsc.subcore_barrier()

        def hop_body(h, _):
            src_off = ybase + h * NP

            def g_desc(bidx, b):
                base = bidx * RB
                return pltpu.make_async_copy(
                    rows_b[b], g.at[pl.ds(nbase + base, RB), :], sem_s[b])

            for bidx in range(NB):
                b = bidx % 2
                if bidx >= 2:
                    g_desc(bidx - 2, b).wait()
                base = bidx * RB
                pltpu.sync_copy(
                    y_sp.at[pl.ds(nbase + base, RB), :], rows_b[b]
                )
                pltpu.async_copy(
                    rows_b[b], g.at[pl.ds(nbase + base, RB), :], sem_s[b])
            for bidx in (NB - 2, NB - 1):
                g_desc(bidx, bidx % 2).wait()
            plsc.subcore_barrier()

            def load_idx(jj, b):
                off = (wid + NS * jj) * CH
                pltpu.async_copy(rows.at[pl.ds(off, CH)], idx_r[b], sem_i[b])
                pltpu.async_copy(cols.at[pl.ds(off, CH)], idx_c[b], sem_c[b])

            def finish_idx(b):
                pltpu.make_async_copy(
                    rows.at[pl.ds(0, CH)], idx_r[b], sem_i[b]).wait()
                pltpu.make_async_copy(
                    cols.at[pl.ds(0, CH)], idx_c[b], sem_c[b]).wait()

            for q in range(4):
                load_idx(q, q)
            for b in range(2):
                finish_idx(b)
                pltpu.async_copy(y_sp.at[idx_r[b]], rows_b[b], sem_g[b])

            def edge_quad(gq, _):
                for u in range(4):
                    j = gq * 4 + u
                    b = u % 2
                    q = u
                    qn = (u + 2) % 4
                    pltpu.make_async_copy(
                        y_sp.at[idx_r[q]], rows_b[b], sem_g[b]).wait()
                    pltpu.async_copy(rows_b[b], g.at[idx_c[q]], sem_s[b],
                                     add=True)
                    pltpu.make_async_copy(
                        rows_b[b], g.at[idx_c[q]], sem_s[b]).wait()

                    @pl.when(j + 4 < CPT)
                    def _():
                        load_idx(j + 4, q)

                    @pl.when(j + 2 < CPT)
                    def _():
                        finish_idx(qn)
                        pltpu.async_copy(y_sp.at[idx_r[qn]], rows_b[b],
                                         sem_g[b])
                return 0
            lax.fori_loop(0, CPT // 4, edge_quad, 0)
            plsc.subcore_barrier()

            scale_slab(g, nbase, t_v, src_off + NP)
            plsc.subcore_barrier()
            return 0
        lax.fori_loop(0, K, hop_body, 0)

    return k(fab, row_p, col_p)


def _tc_linear(YA, YB, WA9, WB9, sinv_col, b):
    nb = NP // 512

    def mm(ya_ref, yb_ref, wa_ref, wb_ref, sv_ref, b_ref, o_ref):
        kk = pl.program_id(1)

        @pl.when(kk == 0)
        def _():
            o_ref[...] = jnp.zeros_like(o_ref)

        o_ref[...] += lax.dot_general(
            ya_ref[0], wa_ref[0], (((1,), (1,)), ((), ())),
            preferred_element_type=jnp.float32,
        ) + lax.dot_general(
            yb_ref[0], wb_ref[0], (((1,), (1,)), ((), ())),
            preferred_element_type=jnp.float32,
        )

        @pl.when(kk == K)
        def _():
            o_ref[...] = o_ref[...] * sv_ref[...] + b_ref[...]

    return pl.pallas_call(
        mm,
        grid=(nb, K + 1),
        in_specs=[
            pl.BlockSpec((1, 512, DH), lambda i, kq: (kq, i, 0)),
            pl.BlockSpec((1, 512, DH), lambda i, kq: (kq, i, 0)),
            pl.BlockSpec((1, OUT_DIM, DH), lambda i, kq: (kq, 0, 0)),
            pl.BlockSpec((1, OUT_DIM, DH), lambda i, kq: (kq, 0, 0)),
            pl.BlockSpec((512, 1), lambda i, kq: (i, 0)),
            pl.BlockSpec((1, OUT_DIM), lambda i, kq: (0, 0)),
        ],
        out_specs=pl.BlockSpec((512, OUT_DIM), lambda i, kq: (i, 0)),
        out_shape=jax.ShapeDtypeStruct((NP, OUT_DIM), jnp.float32),
    )(YA, YB, WA9, WB9, sinv_col, b)


@jax.jit
def kernel(feature, edge_index, W, b):
    fa = jnp.pad(feature[:, :DH], ((0, NP - N), (0, 0)))
    fb = jnp.pad(feature[:, DH:], ((0, NP - N), (0, 0)))
    fab = jnp.concatenate([fa, fb], axis=0)
    row = edge_index[0]
    col = edge_index[1]
    row_p = jnp.pad(row, (0, EP - E))
    col_p = jnp.pad(col, (0, EP - E), constant_values=TRASH)

    Yab, sinv = _sc_propagate(fab, row_p, col_p)
    YA = Yab[:YROWS].reshape(K + 1, NP, DH)
    YB = Yab[YROWS:].reshape(K + 1, NP, DH)

    W9 = jnp.transpose(W.reshape(OUT_DIM, K + 1, D), (1, 0, 2))
    out = _tc_linear(YA, YB, W9[:, :, :DH], W9[:, :, DH:],
                     sinv[:NP].reshape(NP, 1), b.reshape(1, OUT_DIM))
    return out[:N]
